# Initial kernel scaffold; baseline (speedup 1.0000x reference)
import jax
import jax.numpy as jnp
from jax.experimental import pallas as pl


def kernel(A_values, B_values, acd_tar, acd_a, acd_b):
    raise NotImplementedError("write your pallas kernel here")


# speedup vs baseline: 1.5218x; 1.5218x over previous
import jax
import jax.numpy as jnp
from jax import lax
from jax.experimental import pallas as pl
from jax.experimental.pallas import tpu as pltpu
from jax.experimental.pallas import tpu_sc as plsc

NC, NS, L = 2, 16, 16
NW = NC * NS

M_TRIPLES = 320000
D = 128
N_OUT = 320000

T = 400
BPW = N_OUT // (NW * T)
NB = NW * BPW
C = 128
GROUPS = C // L
BND_PAD = 48

def _lane():
    return lax.iota(jnp.int32, L)


def _extract_i32(vec, r):
    return jnp.max(jnp.where(_lane() == r, vec, jnp.int32(-2147483648)))


def _extract_f32(vec, r):
    return jnp.max(jnp.where(_lane() == r, vec, jnp.float32(-jnp.inf)))


def _sc_body(a_hbm, b_hbm, ia_hbm, ib_hbm, tar_hbm, bnd_hbm, out_hbm,
             bnd_v, ia_v, ib_v, tar_v, bg_v, rows_v, acc_v,
             sem_a, sem_b, sem_i):
    wid = lax.axis_index("s") * NC + lax.axis_index("c")
    t_w0 = wid * (BPW * T)

    pltpu.sync_copy(bnd_hbm.at[wid], bnd_v)

    def block_body(i, _):
        t_base = pl.multiple_of(t_w0 + i * T, 8)
        def bnd(idx):
            w0 = pl.multiple_of(idx & jnp.int32(~15), 16)
            vec = bnd_v[pl.ds(w0, L)]
            return jnp.max(jnp.where(_lane() == idx - w0, vec,
                                     jnp.int32(-2147483648)))

        lo = bnd(i)
        hi = bnd(i + 1)
        lo_al = lo & jnp.int32(~15)
        nchunks = (hi - lo_al + (C - 1)) // C

        def zero_body(t, _):
            for c in range(D // L):
                acc_v[t, pl.ds(c * L, L)] = jnp.zeros((L,), jnp.float32)
            return ()
        lax.fori_loop(0, T, zero_body, (), unroll=4)

        def chunk_body(k, _):
            start = pl.multiple_of(
                jnp.minimum(lo_al + k * C, M_TRIPLES - C), 16)
            cp_i = pltpu.async_copy(ia_hbm.at[pl.ds(start, C)], ia_v, sem_i)
            cp_j = pltpu.async_copy(ib_hbm.at[pl.ds(start, C)], ib_v, sem_i)
            cp_t = pltpu.async_copy(tar_hbm.at[pl.ds(start, C)], tar_v, sem_i)
            cp_i.wait(); cp_j.wait(); cp_t.wait()
            cp_a = pltpu.async_copy(a_hbm.at[ia_v], rows_v, sem_a)
            cp_b = pltpu.async_copy(b_hbm.at[ib_v], bg_v, sem_b)
            cp_a.wait(); cp_b.wait()

            vlo = jnp.maximum(lo, lo_al + k * C)
            rot = (_lane() + 1) & jnp.int32(15)
            for g in range(GROUPS):
                jvec = start + g * L + _lane()
                tvec = tar_v[pl.ds(g * L, L)]
                bvec = bg_v[pl.ds(g * L, L)]
                valid = (jvec >= vlo) & (jvec < hi)
                bvec = jnp.where(valid, bvec, jnp.float32(0.0))
                t_loc = jnp.clip(tvec - t_base, 0, T - 1)

                @plsc.parallel_loop(0, L, unroll=4, carry=(t_loc, bvec))
                def row_body(r, car, g=g):
                    tl, bv = car
                    trow = tl[0]
                    b_sp = lax.broadcast(bv[0], (L,))
                    for c in range(D // L):
                        a = rows_v[g * L + r, pl.ds(c * L, L)]
                        plsc.addupdate(acc_v.at[trow, pl.ds(c * L, L)],
                                       a * b_sp)
                    return (tl.at[rot].get(mode="promise_in_bounds"),
                            bv.at[rot].get(mode="promise_in_bounds"))
            return ()

        lax.fori_loop(0, nchunks, chunk_body, ())

        pltpu.sync_copy(acc_v, out_hbm.at[pl.ds(t_base, T), :])
        return ()

    lax.fori_loop(0, BPW, block_body, ())


def kernel(A_values, B_values, acd_tar, acd_a, acd_b):
    edges = (jnp.arange(NB + 1, dtype=jnp.int32) * T).astype(acd_tar.dtype)
    bounds = jnp.searchsorted(acd_tar, edges, side="left").astype(jnp.int32)
    widx = (jnp.arange(NW, dtype=jnp.int32)[:, None] * BPW
            + jnp.arange(BND_PAD, dtype=jnp.int32)[None, :])
    bnd2d = bounds[jnp.clip(widx, 0, NB)]

    mesh = plsc.VectorSubcoreMesh(core_axis_name="c", subcore_axis_name="s")
    f = pl.kernel(
        _sc_body,
        out_type=jax.ShapeDtypeStruct((N_OUT, D), jnp.float32),
        mesh=mesh,
        scratch_types=[
            pltpu.VMEM((BND_PAD,), jnp.int32),
            pltpu.VMEM((C,), jnp.int32),
            pltpu.VMEM((C,), jnp.int32),
            pltpu.VMEM((C,), jnp.int32),
            pltpu.VMEM((C,), jnp.float32),
            pltpu.VMEM((C, D), jnp.float32),
            pltpu.VMEM((T, D), jnp.float32),
            pltpu.SemaphoreType.DMA,
            pltpu.SemaphoreType.DMA,
            pltpu.SemaphoreType.DMA,
        ],
        compiler_params=pltpu.CompilerParams(needs_layout_passes=False),
    )
    return f(A_values, B_values, acd_a, acd_b, acd_tar, bnd2d)
